# baseline (speedup 1.0000x reference)
import functools

import jax
import jax.numpy as jnp
from jax.experimental import pallas as pl

N = 32768
D = 64
K = 1024
TILE_N = 2048


def _kmeans_assign_kernel(x_ref, c_ref, out_ref):
    x = x_ref[...]
    c = c_ref[...]
    x2 = jnp.sum(x * x, axis=1, keepdims=True)
    c2 = jnp.sum(c * c, axis=1)[None, :]
    dot = jax.lax.dot_general(
        x, c, (((1,), (1,)), ((), ())),
        preferred_element_type=jnp.float32)
    dist = x2 - 2.0 * dot + c2
    minval = jnp.min(dist, axis=1, keepdims=True)
    ids = jax.lax.broadcasted_iota(jnp.int32, dist.shape, 1)
    idx = jnp.min(jnp.where(dist == minval, ids, K), axis=1)
    out_ref[...] = idx.astype(jnp.int32)


@jax.jit
def kernel(input, center):
    grid = (N // TILE_N,)
    out = pl.pallas_call(
        _kmeans_assign_kernel,
        grid=grid,
        in_specs=[
            pl.BlockSpec((TILE_N, D), lambda i: (i, 0)),
            pl.BlockSpec((K, D), lambda i: (0, 0)),
        ],
        out_specs=pl.BlockSpec((TILE_N,), lambda i: (i,)),
        out_shape=jax.ShapeDtypeStruct((N,), jnp.int32),
    )(input, center)
    return out


# speedup vs baseline: 2.1500x; 2.1500x over previous
import functools

import jax
import jax.numpy as jnp
from jax.experimental import pallas as pl

N = 32768
D = 64
K = 1024
TILE_N = 2048
CK = 256


def _kmeans_assign_kernel(x_ref, c_ref, out_ref):
    x = x_ref[...]
    c = c_ref[...]
    c2 = jnp.sum(c * c, axis=1, keepdims=True)
    cm = -2.0 * c
    xat = x.T

    state_v = None
    for j in range(K // CK):
        dot = jax.lax.dot_general(
            cm[j * CK:(j + 1) * CK, :], xat, (((1,), (0,)), ((), ())),
            preferred_element_type=jnp.float32)
        s = dot + c2[j * CK:(j + 1) * CK, :]
        for q in range(CK // 8):
            g = j * (CK // 8) + q
            row_v = s[q * 8:(q + 1) * 8, :]
            if state_v is None:
                state_v = row_v
                state_g = jnp.zeros((8, x.shape[0]), jnp.int32)
            else:
                mask = row_v < state_v
                state_v = jnp.minimum(state_v, row_v)
                state_g = jnp.where(mask, jnp.int32(g), state_g)

    sub = jax.lax.broadcasted_iota(jnp.int32, (8, x.shape[0]), 0)
    idx = state_g * 8 + sub
    v = state_v
    for h in (4, 2, 1):
        va, vb = v[:h, :], v[h:2 * h, :]
        ia, ib = idx[:h, :], idx[h:2 * h, :]
        m = (vb < va) | ((vb == va) & (ib < ia))
        v = jnp.where(m, vb, va)
        idx = jnp.where(m, ib, ia)
    out_ref[...] = idx[0, :]


@jax.jit
def kernel(input, center):
    grid = (N // TILE_N,)
    out = pl.pallas_call(
        _kmeans_assign_kernel,
        grid=grid,
        in_specs=[
            pl.BlockSpec((TILE_N, D), lambda i: (i, 0)),
            pl.BlockSpec((K, D), lambda i: (0, 0)),
        ],
        out_specs=pl.BlockSpec((TILE_N,), lambda i: (i,)),
        out_shape=jax.ShapeDtypeStruct((N,), jnp.int32),
    )(input, center)
    return out


# speedup vs baseline: 2.2485x; 1.0458x over previous
import functools

import jax
import jax.numpy as jnp
from jax.experimental import pallas as pl

N = 32768
D = 64
K = 1024
TILE_N = 4096
CK = 256


def _kmeans_assign_kernel(x_ref, c_ref, out_ref):
    x = x_ref[...]
    c = c_ref[...]
    c2 = jnp.sum(c * c, axis=1, keepdims=True)
    cm = -2.0 * c
    xat = x.T

    state_v = None
    for j in range(K // CK):
        dot = jax.lax.dot_general(
            cm[j * CK:(j + 1) * CK, :], xat, (((1,), (0,)), ((), ())),
            preferred_element_type=jnp.float32)
        s = dot + c2[j * CK:(j + 1) * CK, :]
        for q in range(CK // 8):
            g = j * (CK // 8) + q
            row_v = s[q * 8:(q + 1) * 8, :]
            if state_v is None:
                state_v = row_v
                state_g = jnp.zeros((8, x.shape[0]), jnp.int32)
            else:
                mask = row_v < state_v
                state_v = jnp.minimum(state_v, row_v)
                state_g = jnp.where(mask, jnp.int32(g), state_g)

    sub = jax.lax.broadcasted_iota(jnp.int32, (8, x.shape[0]), 0)
    idx = state_g * 8 + sub
    v = state_v
    for h in (4, 2, 1):
        va, vb = v[:h, :], v[h:2 * h, :]
        ia, ib = idx[:h, :], idx[h:2 * h, :]
        m = (vb < va) | ((vb == va) & (ib < ia))
        v = jnp.where(m, vb, va)
        idx = jnp.where(m, ib, ia)
    out_ref[...] = idx[0, :]


@jax.jit
def kernel(input, center):
    grid = (N // TILE_N,)
    out = pl.pallas_call(
        _kmeans_assign_kernel,
        grid=grid,
        in_specs=[
            pl.BlockSpec((TILE_N, D), lambda i: (i, 0)),
            pl.BlockSpec((K, D), lambda i: (0, 0)),
        ],
        out_specs=pl.BlockSpec((TILE_N,), lambda i: (i,)),
        out_shape=jax.ShapeDtypeStruct((N,), jnp.int32),
    )(input, center)
    return out


# speedup vs baseline: 2.2492x; 1.0003x over previous
import functools

import jax
import jax.numpy as jnp
from jax.experimental import pallas as pl
from jax.experimental.pallas import tpu as pltpu

N = 32768
D = 64
K = 1024
TILE_N = 4096
CK = 256


def _kmeans_assign_kernel(x_ref, c_ref, out_ref):
    x = x_ref[...]
    c = c_ref[...]
    c2 = jnp.sum(c * c, axis=1, keepdims=True)
    cm = -2.0 * c
    xat = x.T

    state_v = None
    for j in range(K // CK):
        dot = jax.lax.dot_general(
            cm[j * CK:(j + 1) * CK, :], xat, (((1,), (0,)), ((), ())),
            preferred_element_type=jnp.float32)
        s = dot + c2[j * CK:(j + 1) * CK, :]
        for q in range(CK // 8):
            g = j * (CK // 8) + q
            row_v = s[q * 8:(q + 1) * 8, :]
            if state_v is None:
                state_v = row_v
                state_g = jnp.zeros((8, x.shape[0]), jnp.int32)
            else:
                mask = row_v < state_v
                state_v = jnp.minimum(state_v, row_v)
                state_g = jnp.where(mask, jnp.int32(g), state_g)

    sub = jax.lax.broadcasted_iota(jnp.int32, (8, x.shape[0]), 0)
    idx = state_g * 8 + sub
    v = state_v
    for h in (4, 2, 1):
        va, vb = v[:h, :], v[h:2 * h, :]
        ia, ib = idx[:h, :], idx[h:2 * h, :]
        m = (vb < va) | ((vb == va) & (ib < ia))
        v = jnp.where(m, vb, va)
        idx = jnp.where(m, ib, ia)
    out_ref[...] = idx[0, :]


@jax.jit
def kernel(input, center):
    grid = (N // TILE_N,)
    out = pl.pallas_call(
        _kmeans_assign_kernel,
        grid=grid,
        in_specs=[
            pl.BlockSpec((TILE_N, D), lambda i: (i, 0)),
            pl.BlockSpec((K, D), lambda i: (0, 0)),
        ],
        out_specs=pl.BlockSpec((TILE_N,), lambda i: (i,)),
        out_shape=jax.ShapeDtypeStruct((N,), jnp.int32),
        compiler_params=pltpu.CompilerParams(
            dimension_semantics=("parallel",)),
    )(input, center)
    return out
